# baseline (speedup 1.0000x reference)
import jax
import jax.numpy as jnp
import numpy as np
from jax.experimental import pallas as pl

_B = 2048
_NP = 40
_EP = 120
_NF = 64
_EF = 16
_H = 128
_EH = 64
_ATT = 128
_MAX = 50
_STEPS = 3
_NSUB = 16
_G = _NSUB

_INV_D = 1.0 / np.sqrt(float(_ATT))


def _bf(x):
    return x.astype(jnp.bfloat16)


def _dot(a, b):
    return jax.lax.dot_general(_bf(a), _bf(b), (((1,), (0,)), ((), ())),
                               preferred_element_type=jnp.float32)


def _dotT0(a, b):
    return jax.lax.dot_general(_bf(a), _bf(b), (((0,), (0,)), ((), ())),
                               preferred_element_type=jnp.float32)


def _dotT1(a, b):
    return jax.lax.dot_general(_bf(a), _bf(b), (((1,), (1,)), ((), ())),
                               preferred_element_type=jnp.float32)


def _chunk_body(mol_ref, nodes_ref, edges_ref, src_ref, dst_ref,
                w_ni, b_ni, w_ei, b_ei, wm_n, wm_e, b_msg, wu_h, wu_a, b_upd,
                wk1, bk1, wk2, bk2, wq1, bq1, wq2, bq2, wv1, bv1, wv2, bv2,
                uk1, ubk1, uk2, ubk2, uq1, ubq1, uq2, ubq2,
                wc1, wc1pad, bc1, wc2, bc2,
                out_ref):
    f32 = jnp.float32
    relu = lambda x: jnp.maximum(x, 0.0)
    T = lambda f: tuple(f(i) for i in range(_NSUB))

    def mlp(x, w1, b1, w2, b2):
        h = T(lambda i: relu(_dot(x[i], w1[...]) + b1[...]))
        return T(lambda i: _dot(h[i], w2[...]) + b2[...])

    chunk_base = pl.program_id(0) * (_G * _NP)
    niota = jax.lax.broadcasted_iota(jnp.int32, (_EP, _NP), 1)
    ones_row = jnp.ones((1, _NP), f32)
    v_pad = _dot(relu(bv1[...]), wv2[...]) + bv2[...]

    src = T(lambda i: (src_ref[i:i + 1, :]
                       - (chunk_base + i * _NP)).reshape(_EP, 1))
    dst = T(lambda i: (dst_ref[i:i + 1, :]
                       - (chunk_base + i * _NP)).reshape(_EP, 1))
    oh_src = T(lambda i: (niota == src[i]).astype(jnp.bfloat16))
    oh_dst = T(lambda i: (niota == dst[i]).astype(jnp.bfloat16))

    nh = T(lambda i: relu(_dot(nodes_ref[i * _NP:(i + 1) * _NP, :], w_ni[...])
                          + b_ni[...]))
    ehid = T(lambda i: relu(_dot(edges_ref[i * _EP:(i + 1) * _EP, :], w_ei[...])
                            + b_ei[...]))
    ehc = T(lambda i: _dot(ehid[i], wm_e[...]))
    for _ in range(_STEPS):
        nhw = T(lambda i: _dot(nh[i], wm_n[...]))
        msg = T(lambda i: relu(_dot(oh_src[i], nhw[i]) + ehc[i] + b_msg[...]))
        agg = T(lambda i: _dotT0(oh_dst[i], msg[i]))
        nh = T(lambda i: relu(_dot(nh[i], wu_h[...])
                              + _dot(agg[i], wu_a[...]) + b_upd[...]))

    Km = mlp(nh, wk1, bk1, wk2, bk2)
    Qm = mlp(nh, wq1, bq1, wq2, bq2)
    Vm = mlp(nh, wv1, bv1, wv2, bv2)
    kqt = T(lambda i: _dotT1(Km[i], Qm[i]) * _INV_D)
    m = T(lambda i: jnp.max(kqt[i], axis=1, keepdims=True))
    p = T(lambda i: jnp.exp(kqt[i] - m[i]))
    attn = T(lambda i: p[i] / jnp.sum(p[i], axis=1, keepdims=True))
    corr = T(lambda i: _dot(attn[i], Vm[i]))

    vsum = T(lambda i: _dot(ones_row, Vm[i]))
    corr_pad = T(lambda i: (vsum[i] + float(_MAX - _NP) * v_pad)
                 * (1.0 / _MAX))

    K2 = mlp(corr, uk1, ubk1, uk2, ubk2)
    K2p = mlp(corr_pad, uk1, ubk1, uk2, ubk2)
    Q2 = mlp(T(lambda i: mol_ref[i:i + 1, :]), uq1, ubq1, uq2, ubq2)
    logit_col = T(lambda i: jnp.sum(K2[i] * Q2[i], axis=1, keepdims=True)
                  * _INV_D)
    pad_logit = T(lambda i: jnp.sum(K2p[i] * Q2[i], axis=1, keepdims=True)
                  * _INV_D)

    h = T(lambda i: relu(_dot(ones_row, logit_col[i] * wc1[...])
                         + pad_logit[i] * wc1pad[...] + bc1[...]))
    o = T(lambda i: jax.nn.sigmoid(_dot(h[i], wc2[...]) + bc2[...]))
    for i in range(_NSUB):
        out_ref[i:i + 1, :] = o[i]


def kernel(mol_a_reprs, node_feats, edge_feats, edge_index, params):
    p = params
    f32 = jnp.float32
    row = lambda b: b.reshape(1, -1).astype(f32)

    src2d = edge_index[0].reshape(_B, _EP).astype(jnp.int32)
    dst2d = edge_index[1].reshape(_B, _EP).astype(jnp.int32)

    wq1, bq1, wq2, bq2 = p["Wq"]
    wk1, bk1, wk2, bk2 = p["Wk"]
    wv1, bv1, wv2, bv2 = p["Wv"]
    uk1, ubk1, uk2, ubk2 = p["Uk"]
    uq1, ubq1, uq2, ubq2 = p["Uq"]
    c1, cb1, c2, cb2 = p["Cs"]

    bf = lambda w: w.astype(jnp.bfloat16)
    weights = (
        bf(p["W_ni"]), row(p["b_ni"]), bf(p["W_ei"]), row(p["b_ei"]),
        bf(p["W_msg"][:_H]), bf(p["W_msg"][_H:]), row(p["b_msg"]),
        bf(p["W_upd"][:_H]), bf(p["W_upd"][_H:]), row(p["b_upd"]),
        bf(wk1), row(bk1), bf(wk2), row(bk2),
        bf(wq1), row(bq1), bf(wq2), row(bq2),
        bf(wv1), row(bv1), bf(wv2), row(bv2),
        bf(uk1), row(ubk1), bf(uk2), row(ubk2),
        bf(uq1), row(ubq1), bf(uq2), row(ubq2),
        c1[:_NP].astype(f32),
        jnp.sum(c1[_NP:], axis=0, keepdims=True),
        row(cb1), bf(c2[:, :_NP]), row(cb2[:_NP]),
    )

    grid = (_B // _G,)
    data_specs = [
        pl.BlockSpec((_G, 256), lambda i: (i, 0)),
        pl.BlockSpec((_G * _NP, _NF), lambda i: (i, 0)),
        pl.BlockSpec((_G * _EP, _EF), lambda i: (i, 0)),
        pl.BlockSpec((_G, _EP), lambda i: (i, 0)),
        pl.BlockSpec((_G, _EP), lambda i: (i, 0)),
    ]
    w_specs = [pl.BlockSpec(w.shape, lambda i: (0,) * w.ndim) for w in weights]

    out2d = pl.pallas_call(
        _chunk_body,
        grid=grid,
        in_specs=data_specs + w_specs,
        out_specs=pl.BlockSpec((_G, _NP), lambda i: (i, 0)),
        out_shape=jax.ShapeDtypeStruct((_B, _NP), f32),
    )(mol_a_reprs, node_feats, edge_feats, src2d, dst2d, *weights)
    return out2d.reshape(-1)


# speedup vs baseline: 1.3146x; 1.3146x over previous
import jax
import jax.numpy as jnp
import numpy as np
from jax.experimental import pallas as pl

_B = 2048
_NP = 40
_EP = 120
_NF = 64
_EF = 16
_H = 128
_EH = 64
_ATT = 128
_MAX = 50
_STEPS = 3
_GS = 4
_NSUB = 16
_G = _NSUB * _GS

_INV_D = 1.0 / np.sqrt(float(_ATT))


def _bf(x):
    return x.astype(jnp.bfloat16)


def _dot(a, b):
    return jax.lax.dot_general(_bf(a), _bf(b), (((1,), (0,)), ((), ())),
                               preferred_element_type=jnp.float32)


def _dotT0(a, b):
    return jax.lax.dot_general(_bf(a), _bf(b), (((0,), (0,)), ((), ())),
                               preferred_element_type=jnp.float32)


def _dotT1(a, b):
    return jax.lax.dot_general(_bf(a), _bf(b), (((1,), (1,)), ((), ())),
                               preferred_element_type=jnp.float32)


def _chunk_body(mol_ref, nodes_ref, edges_ref, src_ref, dst_ref,
                w_ni, b_ni, w_ei, b_ei, wm_n, wm_e, b_msg, wu_h, wu_a, b_upd,
                wk1, bk1, wk2, bk2, wq1, bq1, wq2, bq2, wv1, bv1, wv2, bv2,
                uk1, ubk1, uk2, ubk2, uq1, ubq1, uq2, ubq2,
                wc1t, wc1pad, bc1, wc2, bc2,
                out_ref):
    GS = _GS
    NC = GS * _NP
    EC = GS * _EP
    f32 = jnp.float32
    relu = lambda x: jnp.maximum(x, 0.0)
    T = lambda f: tuple(f(i) for i in range(_NSUB))

    def mlp(x, w1, b1, w2, b2):
        h = T(lambda i: relu(_dot(x[i], w1[...]) + b1[...]))
        return T(lambda i: _dot(h[i], w2[...]) + b2[...])

    chunk_base = pl.program_id(0) * (_G * _NP)
    niota = jax.lax.broadcasted_iota(jnp.int32, (GS, _EP, NC), 2)
    same_g = (jax.lax.broadcasted_iota(jnp.int32, (NC, NC), 0) // _NP ==
              jax.lax.broadcasted_iota(jnp.int32, (NC, NC), 1) // _NP)
    ones_bd = (jax.lax.broadcasted_iota(jnp.int32, (GS, NC), 1) // _NP ==
               jax.lax.broadcasted_iota(jnp.int32, (GS, NC), 0)).astype(f32)
    v_pad = _dot(relu(bv1[...]), wv2[...]) + bv2[...]

    src = T(lambda i: (src_ref[i * GS:(i + 1) * GS, :]
                       - (chunk_base + i * NC))[:, :, None])
    dst = T(lambda i: (dst_ref[i * GS:(i + 1) * GS, :]
                       - (chunk_base + i * NC))[:, :, None])
    oh_src = T(lambda i: (niota == src[i]).astype(jnp.bfloat16).reshape(EC, NC))
    oh_dst = T(lambda i: (niota == dst[i]).astype(jnp.bfloat16).reshape(EC, NC))

    nh = T(lambda i: relu(_dot(nodes_ref[i * NC:(i + 1) * NC, :], w_ni[...])
                          + b_ni[...]))
    ehid = T(lambda i: relu(_dot(edges_ref[i * EC:(i + 1) * EC, :], w_ei[...])
                            + b_ei[...]))
    ehc = T(lambda i: _dot(ehid[i], wm_e[...]))
    for _ in range(_STEPS):
        nhw = T(lambda i: _dot(nh[i], wm_n[...]))
        msg = T(lambda i: relu(_dot(oh_src[i], nhw[i]) + ehc[i] + b_msg[...]))
        agg = T(lambda i: _dotT0(oh_dst[i], msg[i]))
        nh = T(lambda i: relu(_dot(nh[i], wu_h[...])
                              + _dot(agg[i], wu_a[...]) + b_upd[...]))

    Km = mlp(nh, wk1, bk1, wk2, bk2)
    Qm = mlp(nh, wq1, bq1, wq2, bq2)
    Vm = mlp(nh, wv1, bv1, wv2, bv2)
    kqt = T(lambda i: _dotT1(Km[i], Qm[i]))
    s = T(lambda i: jnp.where(same_g, kqt[i], -1e9) * _INV_D)
    m = T(lambda i: jnp.max(s[i], axis=1, keepdims=True))
    p = T(lambda i: jnp.exp(s[i] - m[i]))
    attn = T(lambda i: p[i] / jnp.sum(p[i], axis=1, keepdims=True))
    corr = T(lambda i: _dot(attn[i], Vm[i]))

    vsum = T(lambda i: _dot(ones_bd, Vm[i]))
    corr_pad = T(lambda i: (vsum[i] + float(_MAX - _NP) * v_pad)
                 * (1.0 / _MAX))

    K2 = mlp(corr, uk1, ubk1, uk2, ubk2)
    K2p = mlp(corr_pad, uk1, ubk1, uk2, ubk2)
    Q2 = mlp(T(lambda i: mol_ref[i * GS:(i + 1) * GS, :]),
             uq1, ubq1, uq2, ubq2)
    Q2e = T(lambda i: _dotT0(ones_bd, Q2[i]))
    logit_col = T(lambda i: jnp.sum(K2[i] * Q2e[i], axis=1, keepdims=True)
                  * _INV_D)
    pad_logit = T(lambda i: jnp.sum(K2p[i] * Q2[i], axis=1, keepdims=True)
                  * _INV_D)

    h = T(lambda i: relu(_dot(ones_bd, logit_col[i] * wc1t[...])
                         + pad_logit[i] * wc1pad[...] + bc1[...]))
    o = T(lambda i: jax.nn.sigmoid(_dot(h[i], wc2[...]) + bc2[...]))
    for i in range(_NSUB):
        out_ref[i * GS:(i + 1) * GS, :] = o[i]


def kernel(mol_a_reprs, node_feats, edge_feats, edge_index, params):
    p = params
    f32 = jnp.float32
    row = lambda b: b.reshape(1, -1).astype(f32)

    src2d = edge_index[0].reshape(_B, _EP).astype(jnp.int32)
    dst2d = edge_index[1].reshape(_B, _EP).astype(jnp.int32)

    wq1, bq1, wq2, bq2 = p["Wq"]
    wk1, bk1, wk2, bk2 = p["Wk"]
    wv1, bv1, wv2, bv2 = p["Wv"]
    uk1, ubk1, uk2, ubk2 = p["Uk"]
    uq1, ubq1, uq2, ubq2 = p["Uq"]
    c1, cb1, c2, cb2 = p["Cs"]

    bf = lambda w: w.astype(jnp.bfloat16)
    weights = (
        bf(p["W_ni"]), row(p["b_ni"]), bf(p["W_ei"]), row(p["b_ei"]),
        bf(p["W_msg"][:_H]), bf(p["W_msg"][_H:]), row(p["b_msg"]),
        bf(p["W_upd"][:_H]), bf(p["W_upd"][_H:]), row(p["b_upd"]),
        bf(wk1), row(bk1), bf(wk2), row(bk2),
        bf(wq1), row(bq1), bf(wq2), row(bq2),
        bf(wv1), row(bv1), bf(wv2), row(bv2),
        bf(uk1), row(ubk1), bf(uk2), row(ubk2),
        bf(uq1), row(ubq1), bf(uq2), row(ubq2),
        jnp.tile(c1[:_NP], (_GS, 1)),
        jnp.sum(c1[_NP:], axis=0, keepdims=True),
        row(cb1), bf(c2[:, :_NP]), row(cb2[:_NP]),
    )

    grid = (_B // _G,)
    data_specs = [
        pl.BlockSpec((_G, 256), lambda i: (i, 0)),
        pl.BlockSpec((_G * _NP, _NF), lambda i: (i, 0)),
        pl.BlockSpec((_G * _EP, _EF), lambda i: (i, 0)),
        pl.BlockSpec((_G, _EP), lambda i: (i, 0)),
        pl.BlockSpec((_G, _EP), lambda i: (i, 0)),
    ]
    w_specs = [pl.BlockSpec(w.shape, lambda i: (0,) * w.ndim) for w in weights]

    out2d = pl.pallas_call(
        _chunk_body,
        grid=grid,
        in_specs=data_specs + w_specs,
        out_specs=pl.BlockSpec((_G, _NP), lambda i: (i, 0)),
        out_shape=jax.ShapeDtypeStruct((_B, _NP), f32),
    )(mol_a_reprs, node_feats, edge_feats, src2d, dst2d, *weights)
    return out2d.reshape(-1)


# speedup vs baseline: 1.4883x; 1.1322x over previous
import jax
import jax.numpy as jnp
import numpy as np
from jax.experimental import pallas as pl

_B = 2048
_NP = 40
_EP = 120
_NF = 64
_EF = 16
_H = 128
_EH = 64
_ATT = 128
_MAX = 50
_STEPS = 3
_GS = 4
_NSUB = 16
_G = _NSUB * _GS

_INV_D = 1.0 / np.sqrt(float(_ATT))


def _bf(x):
    return x.astype(jnp.bfloat16)


def _dot(a, b):
    return jax.lax.dot_general(_bf(a), _bf(b), (((1,), (0,)), ((), ())),
                               preferred_element_type=jnp.float32)


def _dotT0(a, b):
    return jax.lax.dot_general(_bf(a), _bf(b), (((0,), (0,)), ((), ())),
                               preferred_element_type=jnp.float32)


def _dotT1(a, b):
    return jax.lax.dot_general(_bf(a), _bf(b), (((1,), (1,)), ((), ())),
                               preferred_element_type=jnp.float32)


def _chunk_body(mol_ref, nodes_ref, edges_ref, src_ref, dst_ref,
                w_ni, b_ni, w_ei, b_ei, wm_n, wm_e, b_msg, wu_h, wu_a, b_upd,
                wk1, bk1, wk2, bk2, wq1, bq1, wq2, bq2, wv1, bv1, wv2, bv2,
                uk1, ubk1, uk2, ubk2, uq1, ubq1, uq2, ubq2,
                wc1t, wc1pad, bc1, wc2, bc2,
                out_ref):
    GS = _GS
    NC = GS * _NP
    EC = GS * _EP
    f32 = jnp.float32
    relu = lambda x: jnp.maximum(x, 0.0)
    T = lambda f: tuple(f(i) for i in range(_NSUB))

    def mlp(x, w1, b1, w2, b2):
        h = T(lambda i: relu(_dot(x[i], w1[...]) + b1[...]))
        return T(lambda i: _dot(h[i], w2[...]) + b2[...])

    chunk_base = pl.program_id(0) * (_G * _NP)
    niota = jax.lax.broadcasted_iota(jnp.int32, (GS, _EP, _NP), 2)
    goff = jax.lax.broadcasted_iota(jnp.int32, (GS, 1, 1), 0) * _NP

    def bdot(a, b):
        return jax.lax.dot_general(_bf(a), _bf(b),
                                   (((2,), (1,)), ((0,), (0,))),
                                   preferred_element_type=jnp.float32)

    def bdotT(a, b):
        return jax.lax.dot_general(_bf(a), _bf(b),
                                   (((1,), (1,)), ((0,), (0,))),
                                   preferred_element_type=jnp.float32)
    same_g = (jax.lax.broadcasted_iota(jnp.int32, (NC, NC), 0) // _NP ==
              jax.lax.broadcasted_iota(jnp.int32, (NC, NC), 1) // _NP)
    ones_bd = (jax.lax.broadcasted_iota(jnp.int32, (GS, NC), 1) // _NP ==
               jax.lax.broadcasted_iota(jnp.int32, (GS, NC), 0)).astype(f32)
    v_pad = _dot(relu(bv1[...]), wv2[...]) + bv2[...]

    src = T(lambda i: (src_ref[i * GS:(i + 1) * GS, :]
                       - (chunk_base + i * NC))[:, :, None] - goff)
    dst = T(lambda i: (dst_ref[i * GS:(i + 1) * GS, :]
                       - (chunk_base + i * NC))[:, :, None] - goff)
    oh_src = T(lambda i: (niota == src[i]).astype(jnp.bfloat16))
    oh_dst = T(lambda i: (niota == dst[i]).astype(jnp.bfloat16))

    nh = T(lambda i: relu(_dot(nodes_ref[i * NC:(i + 1) * NC, :], w_ni[...])
                          + b_ni[...]))
    ehid = T(lambda i: relu(_dot(edges_ref[i * EC:(i + 1) * EC, :], w_ei[...])
                            + b_ei[...]))
    ehc = T(lambda i: _dot(ehid[i], wm_e[...]))
    for _ in range(_STEPS):
        nhw = T(lambda i: _dot(nh[i], wm_n[...]))
        msg = T(lambda i: relu(
            bdot(oh_src[i], nhw[i].reshape(GS, _NP, _H)).reshape(EC, _H)
            + ehc[i] + b_msg[...]))
        agg = T(lambda i: bdotT(oh_dst[i],
                                msg[i].reshape(GS, _EP, _H)).reshape(NC, _H))
        nh = T(lambda i: relu(_dot(nh[i], wu_h[...])
                              + _dot(agg[i], wu_a[...]) + b_upd[...]))

    Km = mlp(nh, wk1, bk1, wk2, bk2)
    Qm = mlp(nh, wq1, bq1, wq2, bq2)
    Vm = mlp(nh, wv1, bv1, wv2, bv2)
    kqt = T(lambda i: _dotT1(Km[i], Qm[i]))
    s = T(lambda i: jnp.where(same_g, kqt[i], -1e9) * _INV_D)
    m = T(lambda i: jnp.max(s[i], axis=1, keepdims=True))
    p = T(lambda i: jnp.exp(s[i] - m[i]))
    attn = T(lambda i: p[i] / jnp.sum(p[i], axis=1, keepdims=True))
    corr = T(lambda i: _dot(attn[i], Vm[i]))

    vsum = T(lambda i: _dot(ones_bd, Vm[i]))
    corr_pad = T(lambda i: (vsum[i] + float(_MAX - _NP) * v_pad)
                 * (1.0 / _MAX))

    K2 = mlp(corr, uk1, ubk1, uk2, ubk2)
    K2p = mlp(corr_pad, uk1, ubk1, uk2, ubk2)
    Q2 = mlp(T(lambda i: mol_ref[i * GS:(i + 1) * GS, :]),
             uq1, ubq1, uq2, ubq2)
    Q2e = T(lambda i: _dotT0(ones_bd, Q2[i]))
    logit_col = T(lambda i: jnp.sum(K2[i] * Q2e[i], axis=1, keepdims=True)
                  * _INV_D)
    pad_logit = T(lambda i: jnp.sum(K2p[i] * Q2[i], axis=1, keepdims=True)
                  * _INV_D)

    h = T(lambda i: relu(_dot(ones_bd, logit_col[i] * wc1t[...])
                         + pad_logit[i] * wc1pad[...] + bc1[...]))
    o = T(lambda i: jax.nn.sigmoid(_dot(h[i], wc2[...]) + bc2[...]))
    for i in range(_NSUB):
        out_ref[i * GS:(i + 1) * GS, :] = o[i]


def kernel(mol_a_reprs, node_feats, edge_feats, edge_index, params):
    p = params
    f32 = jnp.float32
    row = lambda b: b.reshape(1, -1).astype(f32)

    src2d = edge_index[0].reshape(_B, _EP).astype(jnp.int32)
    dst2d = edge_index[1].reshape(_B, _EP).astype(jnp.int32)

    wq1, bq1, wq2, bq2 = p["Wq"]
    wk1, bk1, wk2, bk2 = p["Wk"]
    wv1, bv1, wv2, bv2 = p["Wv"]
    uk1, ubk1, uk2, ubk2 = p["Uk"]
    uq1, ubq1, uq2, ubq2 = p["Uq"]
    c1, cb1, c2, cb2 = p["Cs"]

    bf = lambda w: w.astype(jnp.bfloat16)
    weights = (
        bf(p["W_ni"]), row(p["b_ni"]), bf(p["W_ei"]), row(p["b_ei"]),
        bf(p["W_msg"][:_H]), bf(p["W_msg"][_H:]), row(p["b_msg"]),
        bf(p["W_upd"][:_H]), bf(p["W_upd"][_H:]), row(p["b_upd"]),
        bf(wk1), row(bk1), bf(wk2), row(bk2),
        bf(wq1), row(bq1), bf(wq2), row(bq2),
        bf(wv1), row(bv1), bf(wv2), row(bv2),
        bf(uk1), row(ubk1), bf(uk2), row(ubk2),
        bf(uq1), row(ubq1), bf(uq2), row(ubq2),
        jnp.tile(c1[:_NP], (_GS, 1)),
        jnp.sum(c1[_NP:], axis=0, keepdims=True),
        row(cb1), bf(c2[:, :_NP]), row(cb2[:_NP]),
    )

    grid = (_B // _G,)
    data_specs = [
        pl.BlockSpec((_G, 256), lambda i: (i, 0)),
        pl.BlockSpec((_G * _NP, _NF), lambda i: (i, 0)),
        pl.BlockSpec((_G * _EP, _EF), lambda i: (i, 0)),
        pl.BlockSpec((_G, _EP), lambda i: (i, 0)),
        pl.BlockSpec((_G, _EP), lambda i: (i, 0)),
    ]
    w_specs = [pl.BlockSpec(w.shape, lambda i: (0,) * w.ndim) for w in weights]

    out2d = pl.pallas_call(
        _chunk_body,
        grid=grid,
        in_specs=data_specs + w_specs,
        out_specs=pl.BlockSpec((_G, _NP), lambda i: (i, 0)),
        out_shape=jax.ShapeDtypeStruct((_B, _NP), f32),
    )(mol_a_reprs, node_feats, edge_feats, src2d, dst2d, *weights)
    return out2d.reshape(-1)


# speedup vs baseline: 1.5306x; 1.0284x over previous
import jax
import jax.numpy as jnp
import numpy as np
from jax.experimental import pallas as pl

_B = 2048
_NP = 40
_EP = 120
_NF = 64
_EF = 16
_H = 128
_EH = 64
_ATT = 128
_MAX = 50
_STEPS = 3
_GS = 4
_NSUB = 16
_G = _NSUB * _GS

_INV_D = 1.0 / np.sqrt(float(_ATT))


def _bf(x):
    return x.astype(jnp.bfloat16)


def _dot(a, b):
    return jax.lax.dot_general(_bf(a), _bf(b), (((1,), (0,)), ((), ())),
                               preferred_element_type=jnp.float32)


def _dotT0(a, b):
    return jax.lax.dot_general(_bf(a), _bf(b), (((0,), (0,)), ((), ())),
                               preferred_element_type=jnp.float32)


def _dotT1(a, b):
    return jax.lax.dot_general(_bf(a), _bf(b), (((1,), (1,)), ((), ())),
                               preferred_element_type=jnp.float32)


def _chunk_body(mol_ref, nodes_ref, edges_ref, src_ref, dst_ref,
                w_ni, b_ni, w_ei, b_ei, wm_n, wm_e, b_msg, wu_h, wu_a, b_upd,
                wk1, bk1, wk2, bk2, wq1, bq1, wq2, bq2, wv1, bv1, wv2, bv2,
                uk1, ubk1, uk2, ubk2, uq1, ubq1, uq2, ubq2,
                wc1t, wc1pad, bc1, wc2, bc2,
                out_ref):
    GS = _GS
    NC = GS * _NP
    EC = GS * _EP
    f32 = jnp.float32
    relu = lambda x: jnp.maximum(x, 0.0)
    T = lambda f: tuple(f(i) for i in range(_NSUB))

    def mlp(x, w1, b1, w2, b2):
        h = T(lambda i: relu(_dot(x[i], w1[...]) + b1[...]))
        return T(lambda i: _dot(h[i], w2[...]) + b2[...])

    chunk_base = pl.program_id(0) * (_G * _NP)
    niota = jax.lax.broadcasted_iota(jnp.int32, (GS, _EP, _NP), 2)
    goff = jax.lax.broadcasted_iota(jnp.int32, (GS, 1, 1), 0) * _NP

    def bdot(a, b):
        return jax.lax.dot_general(_bf(a), _bf(b),
                                   (((2,), (1,)), ((0,), (0,))),
                                   preferred_element_type=jnp.float32)

    def bdotT(a, b):
        return jax.lax.dot_general(_bf(a), _bf(b),
                                   (((1,), (1,)), ((0,), (0,))),
                                   preferred_element_type=jnp.float32)

    def bdotR(a, b):
        return jax.lax.dot_general(_bf(a), _bf(b),
                                   (((2,), (2,)), ((0,), (0,))),
                                   preferred_element_type=jnp.float32)
    ones_bd = (jax.lax.broadcasted_iota(jnp.int32, (GS, NC), 1) // _NP ==
               jax.lax.broadcasted_iota(jnp.int32, (GS, NC), 0)).astype(f32)
    v_pad = _dot(relu(bv1[...]), wv2[...]) + bv2[...]

    src = T(lambda i: (src_ref[i * GS:(i + 1) * GS, :]
                       - (chunk_base + i * NC))[:, :, None] - goff)
    dst = T(lambda i: (dst_ref[i * GS:(i + 1) * GS, :]
                       - (chunk_base + i * NC))[:, :, None] - goff)
    oh_src = T(lambda i: (niota == src[i]).astype(jnp.bfloat16))
    oh_dst = T(lambda i: (niota == dst[i]).astype(jnp.bfloat16))

    nh = T(lambda i: relu(_dot(nodes_ref[i * NC:(i + 1) * NC, :], w_ni[...])
                          + b_ni[...]))
    ehid = T(lambda i: relu(_dot(edges_ref[i * EC:(i + 1) * EC, :], w_ei[...])
                            + b_ei[...]))
    ehc = T(lambda i: _dot(ehid[i], wm_e[...]))
    for _ in range(_STEPS):
        nhw = T(lambda i: _dot(nh[i], wm_n[...]))
        msg = T(lambda i: relu(
            bdot(oh_src[i], nhw[i].reshape(GS, _NP, _H)).reshape(EC, _H)
            + ehc[i] + b_msg[...]))
        agg = T(lambda i: bdotT(oh_dst[i],
                                msg[i].reshape(GS, _EP, _H)).reshape(NC, _H))
        nh = T(lambda i: relu(_dot(nh[i], wu_h[...])
                              + _dot(agg[i], wu_a[...]) + b_upd[...]))

    Km = mlp(nh, wk1, bk1, wk2, bk2)
    Qm = mlp(nh, wq1, bq1, wq2, bq2)
    Vm = mlp(nh, wv1, bv1, wv2, bv2)
    Vm3 = T(lambda i: Vm[i].reshape(GS, _NP, _ATT))
    kqt = T(lambda i: bdotR(Km[i].reshape(GS, _NP, _ATT),
                            Qm[i].reshape(GS, _NP, _ATT)) * _INV_D)
    m = T(lambda i: jnp.max(kqt[i], axis=2, keepdims=True))
    p = T(lambda i: jnp.exp(kqt[i] - m[i]))
    attn = T(lambda i: p[i] / jnp.sum(p[i], axis=2, keepdims=True))
    corr = T(lambda i: bdot(attn[i], Vm3[i]).reshape(NC, _ATT))

    vsum = T(lambda i: _dot(ones_bd, Vm[i]))
    corr_pad = T(lambda i: (vsum[i] + float(_MAX - _NP) * v_pad)
                 * (1.0 / _MAX))

    K2 = mlp(corr, uk1, ubk1, uk2, ubk2)
    K2p = mlp(corr_pad, uk1, ubk1, uk2, ubk2)
    Q2 = mlp(T(lambda i: mol_ref[i * GS:(i + 1) * GS, :]),
             uq1, ubq1, uq2, ubq2)
    Q2e = T(lambda i: _dotT0(ones_bd, Q2[i]))
    logit_col = T(lambda i: jnp.sum(K2[i] * Q2e[i], axis=1, keepdims=True)
                  * _INV_D)
    pad_logit = T(lambda i: jnp.sum(K2p[i] * Q2[i], axis=1, keepdims=True)
                  * _INV_D)

    h = T(lambda i: relu(_dot(ones_bd, logit_col[i] * wc1t[...])
                         + pad_logit[i] * wc1pad[...] + bc1[...]))
    o = T(lambda i: jax.nn.sigmoid(_dot(h[i], wc2[...]) + bc2[...]))
    for i in range(_NSUB):
        out_ref[i * GS:(i + 1) * GS, :] = o[i]


def kernel(mol_a_reprs, node_feats, edge_feats, edge_index, params):
    p = params
    f32 = jnp.float32
    row = lambda b: b.reshape(1, -1).astype(f32)

    src2d = edge_index[0].reshape(_B, _EP).astype(jnp.int32)
    dst2d = edge_index[1].reshape(_B, _EP).astype(jnp.int32)

    wq1, bq1, wq2, bq2 = p["Wq"]
    wk1, bk1, wk2, bk2 = p["Wk"]
    wv1, bv1, wv2, bv2 = p["Wv"]
    uk1, ubk1, uk2, ubk2 = p["Uk"]
    uq1, ubq1, uq2, ubq2 = p["Uq"]
    c1, cb1, c2, cb2 = p["Cs"]

    bf = lambda w: w.astype(jnp.bfloat16)
    weights = (
        bf(p["W_ni"]), row(p["b_ni"]), bf(p["W_ei"]), row(p["b_ei"]),
        bf(p["W_msg"][:_H]), bf(p["W_msg"][_H:]), row(p["b_msg"]),
        bf(p["W_upd"][:_H]), bf(p["W_upd"][_H:]), row(p["b_upd"]),
        bf(wk1), row(bk1), bf(wk2), row(bk2),
        bf(wq1), row(bq1), bf(wq2), row(bq2),
        bf(wv1), row(bv1), bf(wv2), row(bv2),
        bf(uk1), row(ubk1), bf(uk2), row(ubk2),
        bf(uq1), row(ubq1), bf(uq2), row(ubq2),
        jnp.tile(c1[:_NP], (_GS, 1)),
        jnp.sum(c1[_NP:], axis=0, keepdims=True),
        row(cb1), bf(c2[:, :_NP]), row(cb2[:_NP]),
    )

    grid = (_B // _G,)
    data_specs = [
        pl.BlockSpec((_G, 256), lambda i: (i, 0)),
        pl.BlockSpec((_G * _NP, _NF), lambda i: (i, 0)),
        pl.BlockSpec((_G * _EP, _EF), lambda i: (i, 0)),
        pl.BlockSpec((_G, _EP), lambda i: (i, 0)),
        pl.BlockSpec((_G, _EP), lambda i: (i, 0)),
    ]
    w_specs = [pl.BlockSpec(w.shape, lambda i: (0,) * w.ndim) for w in weights]

    out2d = pl.pallas_call(
        _chunk_body,
        grid=grid,
        in_specs=data_specs + w_specs,
        out_specs=pl.BlockSpec((_G, _NP), lambda i: (i, 0)),
        out_shape=jax.ShapeDtypeStruct((_B, _NP), f32),
    )(mol_a_reprs, node_feats, edge_feats, src2d, dst2d, *weights)
    return out2d.reshape(-1)


# speedup vs baseline: 1.5629x; 1.0211x over previous
import jax
import jax.numpy as jnp
import numpy as np
from jax.experimental import pallas as pl

_B = 2048
_NP = 40
_EP = 120
_NF = 64
_EF = 16
_H = 128
_EH = 64
_ATT = 128
_MAX = 50
_STEPS = 3
_GS = 8
_NSUB = 8
_G = _NSUB * _GS

_INV_D = 1.0 / np.sqrt(float(_ATT))


def _bf(x):
    return x.astype(jnp.bfloat16)


def _dot(a, b):
    return jax.lax.dot_general(_bf(a), _bf(b), (((1,), (0,)), ((), ())),
                               preferred_element_type=jnp.float32)


def _dotT0(a, b):
    return jax.lax.dot_general(_bf(a), _bf(b), (((0,), (0,)), ((), ())),
                               preferred_element_type=jnp.float32)


def _dotT1(a, b):
    return jax.lax.dot_general(_bf(a), _bf(b), (((1,), (1,)), ((), ())),
                               preferred_element_type=jnp.float32)


def _chunk_body(mol_ref, nodes_ref, edges_ref, src_ref, dst_ref,
                w_ni, b_ni, w_ei, b_ei, wm_n, wm_e, b_msg, wu_h, wu_a, b_upd,
                wk1, bk1, wk2, bk2, wq1, bq1, wq2, bq2, wv1, bv1, wv2, bv2,
                uk1, ubk1, uk2, ubk2, uq1, ubq1, uq2, ubq2,
                wc1t, wc1pad, bc1, wc2, bc2,
                out_ref):
    GS = _GS
    NC = GS * _NP
    EC = GS * _EP
    f32 = jnp.float32
    relu = lambda x: jnp.maximum(x, 0.0)
    T = lambda f: tuple(f(i) for i in range(_NSUB))

    def mlp(x, w1, b1, w2, b2):
        h = T(lambda i: relu(_dot(x[i], w1[...]) + b1[...]))
        return T(lambda i: _dot(h[i], w2[...]) + b2[...])

    chunk_base = pl.program_id(0) * (_G * _NP)
    niota = jax.lax.broadcasted_iota(jnp.int32, (GS, _EP, _NP), 2)
    goff = jax.lax.broadcasted_iota(jnp.int32, (GS, 1, 1), 0) * _NP

    def bdot(a, b):
        return jax.lax.dot_general(_bf(a), _bf(b),
                                   (((2,), (1,)), ((0,), (0,))),
                                   preferred_element_type=jnp.float32)

    def bdotT(a, b):
        return jax.lax.dot_general(_bf(a), _bf(b),
                                   (((1,), (1,)), ((0,), (0,))),
                                   preferred_element_type=jnp.float32)

    def bdotR(a, b):
        return jax.lax.dot_general(_bf(a), _bf(b),
                                   (((2,), (2,)), ((0,), (0,))),
                                   preferred_element_type=jnp.float32)
    ones_bd = (jax.lax.broadcasted_iota(jnp.int32, (GS, NC), 1) // _NP ==
               jax.lax.broadcasted_iota(jnp.int32, (GS, NC), 0)).astype(f32)
    v_pad = _dot(relu(bv1[...]), wv2[...]) + bv2[...]

    src = T(lambda i: (src_ref[i * GS:(i + 1) * GS, :]
                       - (chunk_base + i * NC))[:, :, None] - goff)
    dst = T(lambda i: (dst_ref[i * GS:(i + 1) * GS, :]
                       - (chunk_base + i * NC))[:, :, None] - goff)
    oh_src = T(lambda i: (niota == src[i]).astype(jnp.bfloat16))
    oh_dst = T(lambda i: (niota == dst[i]).astype(jnp.bfloat16))

    nh = T(lambda i: relu(_dot(nodes_ref[i * NC:(i + 1) * NC, :], w_ni[...])
                          + b_ni[...]))
    ehid = T(lambda i: relu(_dot(edges_ref[i * EC:(i + 1) * EC, :], w_ei[...])
                            + b_ei[...]))
    ehc = T(lambda i: _dot(ehid[i], wm_e[...]))
    for _ in range(_STEPS):
        nhw = T(lambda i: _dot(nh[i], wm_n[...]))
        msg = T(lambda i: relu(
            bdot(oh_src[i], nhw[i].reshape(GS, _NP, _H)).reshape(EC, _H)
            + ehc[i] + b_msg[...]))
        agg = T(lambda i: bdotT(oh_dst[i],
                                msg[i].reshape(GS, _EP, _H)).reshape(NC, _H))
        nh = T(lambda i: relu(_dot(nh[i], wu_h[...])
                              + _dot(agg[i], wu_a[...]) + b_upd[...]))

    Km = mlp(nh, wk1, bk1, wk2, bk2)
    Qm = mlp(nh, wq1, bq1, wq2, bq2)
    Vm = mlp(nh, wv1, bv1, wv2, bv2)
    Vm3 = T(lambda i: Vm[i].reshape(GS, _NP, _ATT))
    kqt = T(lambda i: bdotR(Km[i].reshape(GS, _NP, _ATT),
                            Qm[i].reshape(GS, _NP, _ATT)) * _INV_D)
    m = T(lambda i: jnp.max(kqt[i], axis=2, keepdims=True))
    p = T(lambda i: jnp.exp(kqt[i] - m[i]))
    attn = T(lambda i: p[i] / jnp.sum(p[i], axis=2, keepdims=True))
    corr = T(lambda i: bdot(attn[i], Vm3[i]).reshape(NC, _ATT))

    vsum = T(lambda i: _dot(ones_bd, Vm[i]))
    corr_pad = T(lambda i: (vsum[i] + float(_MAX - _NP) * v_pad)
                 * (1.0 / _MAX))

    K2 = mlp(corr, uk1, ubk1, uk2, ubk2)
    K2p = mlp(corr_pad, uk1, ubk1, uk2, ubk2)
    Q2 = mlp(T(lambda i: mol_ref[i * GS:(i + 1) * GS, :]),
             uq1, ubq1, uq2, ubq2)
    Q2e = T(lambda i: _dotT0(ones_bd, Q2[i]))
    logit_col = T(lambda i: jnp.sum(K2[i] * Q2e[i], axis=1, keepdims=True)
                  * _INV_D)
    pad_logit = T(lambda i: jnp.sum(K2p[i] * Q2[i], axis=1, keepdims=True)
                  * _INV_D)

    h = T(lambda i: relu(_dot(ones_bd, logit_col[i] * wc1t[...])
                         + pad_logit[i] * wc1pad[...] + bc1[...]))
    o = T(lambda i: jax.nn.sigmoid(_dot(h[i], wc2[...]) + bc2[...]))
    for i in range(_NSUB):
        out_ref[i * GS:(i + 1) * GS, :] = o[i]


def kernel(mol_a_reprs, node_feats, edge_feats, edge_index, params):
    p = params
    f32 = jnp.float32
    row = lambda b: b.reshape(1, -1).astype(f32)

    src2d = edge_index[0].reshape(_B, _EP).astype(jnp.int32)
    dst2d = edge_index[1].reshape(_B, _EP).astype(jnp.int32)

    wq1, bq1, wq2, bq2 = p["Wq"]
    wk1, bk1, wk2, bk2 = p["Wk"]
    wv1, bv1, wv2, bv2 = p["Wv"]
    uk1, ubk1, uk2, ubk2 = p["Uk"]
    uq1, ubq1, uq2, ubq2 = p["Uq"]
    c1, cb1, c2, cb2 = p["Cs"]

    bf = lambda w: w.astype(jnp.bfloat16)
    weights = (
        bf(p["W_ni"]), row(p["b_ni"]), bf(p["W_ei"]), row(p["b_ei"]),
        bf(p["W_msg"][:_H]), bf(p["W_msg"][_H:]), row(p["b_msg"]),
        bf(p["W_upd"][:_H]), bf(p["W_upd"][_H:]), row(p["b_upd"]),
        bf(wk1), row(bk1), bf(wk2), row(bk2),
        bf(wq1), row(bq1), bf(wq2), row(bq2),
        bf(wv1), row(bv1), bf(wv2), row(bv2),
        bf(uk1), row(ubk1), bf(uk2), row(ubk2),
        bf(uq1), row(ubq1), bf(uq2), row(ubq2),
        jnp.tile(c1[:_NP], (_GS, 1)),
        jnp.sum(c1[_NP:], axis=0, keepdims=True),
        row(cb1), bf(c2[:, :_NP]), row(cb2[:_NP]),
    )

    grid = (_B // _G,)
    data_specs = [
        pl.BlockSpec((_G, 256), lambda i: (i, 0)),
        pl.BlockSpec((_G * _NP, _NF), lambda i: (i, 0)),
        pl.BlockSpec((_G * _EP, _EF), lambda i: (i, 0)),
        pl.BlockSpec((_G, _EP), lambda i: (i, 0)),
        pl.BlockSpec((_G, _EP), lambda i: (i, 0)),
    ]
    w_specs = [pl.BlockSpec(w.shape, lambda i: (0,) * w.ndim) for w in weights]

    out2d = pl.pallas_call(
        _chunk_body,
        grid=grid,
        in_specs=data_specs + w_specs,
        out_specs=pl.BlockSpec((_G, _NP), lambda i: (i, 0)),
        out_shape=jax.ShapeDtypeStruct((_B, _NP), f32),
    )(mol_a_reprs, node_feats, edge_feats, src2d, dst2d, *weights)
    return out2d.reshape(-1)


# speedup vs baseline: 1.6070x; 1.0282x over previous
import jax
import jax.numpy as jnp
import numpy as np
from jax.experimental import pallas as pl

_B = 2048
_NP = 40
_EP = 120
_NF = 64
_EF = 16
_H = 128
_EH = 64
_ATT = 128
_MAX = 50
_STEPS = 3
_GS = 8
_NSUB = 16
_G = _NSUB * _GS

_INV_D = 1.0 / np.sqrt(float(_ATT))


def _bf(x):
    return x.astype(jnp.bfloat16)


def _dot(a, b):
    return jax.lax.dot_general(_bf(a), _bf(b), (((1,), (0,)), ((), ())),
                               preferred_element_type=jnp.float32)


def _dotT0(a, b):
    return jax.lax.dot_general(_bf(a), _bf(b), (((0,), (0,)), ((), ())),
                               preferred_element_type=jnp.float32)


def _dotT1(a, b):
    return jax.lax.dot_general(_bf(a), _bf(b), (((1,), (1,)), ((), ())),
                               preferred_element_type=jnp.float32)


def _chunk_body(mol_ref, nodes_ref, edges_ref, src_ref, dst_ref,
                w_ni, b_ni, w_ei, b_ei, wm_n, wm_e, b_msg, wu_h, wu_a, b_upd,
                wk1, bk1, wk2, bk2, wq1, bq1, wq2, bq2, wv1, bv1, wv2, bv2,
                uk1, ubk1, uk2, ubk2, uq1, ubq1, uq2, ubq2,
                wc1t, wc1pad, bc1, wc2, bc2,
                out_ref):
    GS = _GS
    NC = GS * _NP
    EC = GS * _EP
    f32 = jnp.float32
    relu = lambda x: jnp.maximum(x, 0.0)
    T = lambda f: tuple(f(i) for i in range(_NSUB))

    def mlp(x, w1, b1, w2, b2):
        h = T(lambda i: relu(_dot(x[i], w1[...]) + b1[...]))
        return T(lambda i: _dot(h[i], w2[...]) + b2[...])

    chunk_base = pl.program_id(0) * (_G * _NP)
    niota = jax.lax.broadcasted_iota(jnp.int32, (GS, _EP, _NP), 2)
    goff = jax.lax.broadcasted_iota(jnp.int32, (GS, 1, 1), 0) * _NP

    def bdot(a, b):
        return jax.lax.dot_general(_bf(a), _bf(b),
                                   (((2,), (1,)), ((0,), (0,))),
                                   preferred_element_type=jnp.float32)

    def bdotT(a, b):
        return jax.lax.dot_general(_bf(a), _bf(b),
                                   (((1,), (1,)), ((0,), (0,))),
                                   preferred_element_type=jnp.float32)

    def bdotR(a, b):
        return jax.lax.dot_general(_bf(a), _bf(b),
                                   (((2,), (2,)), ((0,), (0,))),
                                   preferred_element_type=jnp.float32)
    ones_bd = (jax.lax.broadcasted_iota(jnp.int32, (GS, NC), 1) // _NP ==
               jax.lax.broadcasted_iota(jnp.int32, (GS, NC), 0)).astype(f32)
    v_pad = _dot(relu(bv1[...]), wv2[...]) + bv2[...]

    src = T(lambda i: (src_ref[i * GS:(i + 1) * GS, :]
                       - (chunk_base + i * NC))[:, :, None] - goff)
    dst = T(lambda i: (dst_ref[i * GS:(i + 1) * GS, :]
                       - (chunk_base + i * NC))[:, :, None] - goff)
    oh_src = T(lambda i: (niota == src[i]).astype(jnp.bfloat16))
    oh_dst = T(lambda i: (niota == dst[i]).astype(jnp.bfloat16))

    nh = T(lambda i: relu(_dot(nodes_ref[i * NC:(i + 1) * NC, :], w_ni[...])
                          + b_ni[...]))
    ehid = T(lambda i: relu(_dot(edges_ref[i * EC:(i + 1) * EC, :], w_ei[...])
                            + b_ei[...]))
    ehc = T(lambda i: _dot(ehid[i], wm_e[...]))
    for _ in range(_STEPS):
        nhw = T(lambda i: _dot(nh[i], wm_n[...]))
        msg = T(lambda i: relu(
            bdot(oh_src[i], nhw[i].reshape(GS, _NP, _H)).reshape(EC, _H)
            + ehc[i] + b_msg[...]))
        agg = T(lambda i: bdotT(oh_dst[i],
                                msg[i].reshape(GS, _EP, _H)).reshape(NC, _H))
        nh = T(lambda i: relu(_dot(nh[i], wu_h[...])
                              + _dot(agg[i], wu_a[...]) + b_upd[...]))

    Km = mlp(nh, wk1, bk1, wk2, bk2)
    Qm = mlp(nh, wq1, bq1, wq2, bq2)
    Vm = mlp(nh, wv1, bv1, wv2, bv2)
    Vm3 = T(lambda i: Vm[i].reshape(GS, _NP, _ATT))
    kqt = T(lambda i: bdotR(Km[i].reshape(GS, _NP, _ATT),
                            Qm[i].reshape(GS, _NP, _ATT)) * _INV_D)
    m = T(lambda i: jnp.max(kqt[i], axis=2, keepdims=True))
    p = T(lambda i: jnp.exp(kqt[i] - m[i]))
    attn = T(lambda i: p[i] / jnp.sum(p[i], axis=2, keepdims=True))
    corr = T(lambda i: bdot(attn[i], Vm3[i]).reshape(NC, _ATT))

    vsum = T(lambda i: _dot(ones_bd, Vm[i]))
    corr_pad = T(lambda i: (vsum[i] + float(_MAX - _NP) * v_pad)
                 * (1.0 / _MAX))

    K2 = mlp(corr, uk1, ubk1, uk2, ubk2)
    K2p = mlp(corr_pad, uk1, ubk1, uk2, ubk2)
    Q2 = mlp(T(lambda i: mol_ref[i * GS:(i + 1) * GS, :]),
             uq1, ubq1, uq2, ubq2)
    Q2e = T(lambda i: _dotT0(ones_bd, Q2[i]))
    logit_col = T(lambda i: jnp.sum(K2[i] * Q2e[i], axis=1, keepdims=True)
                  * _INV_D)
    pad_logit = T(lambda i: jnp.sum(K2p[i] * Q2[i], axis=1, keepdims=True)
                  * _INV_D)

    h = T(lambda i: relu(_dot(ones_bd, logit_col[i] * wc1t[...])
                         + pad_logit[i] * wc1pad[...] + bc1[...]))
    o = T(lambda i: jax.nn.sigmoid(_dot(h[i], wc2[...]) + bc2[...]))
    for i in range(_NSUB):
        out_ref[i * GS:(i + 1) * GS, :] = o[i]


def kernel(mol_a_reprs, node_feats, edge_feats, edge_index, params):
    p = params
    f32 = jnp.float32
    row = lambda b: b.reshape(1, -1).astype(f32)

    src2d = edge_index[0].reshape(_B, _EP).astype(jnp.int32)
    dst2d = edge_index[1].reshape(_B, _EP).astype(jnp.int32)

    wq1, bq1, wq2, bq2 = p["Wq"]
    wk1, bk1, wk2, bk2 = p["Wk"]
    wv1, bv1, wv2, bv2 = p["Wv"]
    uk1, ubk1, uk2, ubk2 = p["Uk"]
    uq1, ubq1, uq2, ubq2 = p["Uq"]
    c1, cb1, c2, cb2 = p["Cs"]

    bf = lambda w: w.astype(jnp.bfloat16)
    weights = (
        bf(p["W_ni"]), row(p["b_ni"]), bf(p["W_ei"]), row(p["b_ei"]),
        bf(p["W_msg"][:_H]), bf(p["W_msg"][_H:]), row(p["b_msg"]),
        bf(p["W_upd"][:_H]), bf(p["W_upd"][_H:]), row(p["b_upd"]),
        bf(wk1), row(bk1), bf(wk2), row(bk2),
        bf(wq1), row(bq1), bf(wq2), row(bq2),
        bf(wv1), row(bv1), bf(wv2), row(bv2),
        bf(uk1), row(ubk1), bf(uk2), row(ubk2),
        bf(uq1), row(ubq1), bf(uq2), row(ubq2),
        jnp.tile(c1[:_NP], (_GS, 1)),
        jnp.sum(c1[_NP:], axis=0, keepdims=True),
        row(cb1), bf(c2[:, :_NP]), row(cb2[:_NP]),
    )

    grid = (_B // _G,)
    data_specs = [
        pl.BlockSpec((_G, 256), lambda i: (i, 0)),
        pl.BlockSpec((_G * _NP, _NF), lambda i: (i, 0)),
        pl.BlockSpec((_G * _EP, _EF), lambda i: (i, 0)),
        pl.BlockSpec((_G, _EP), lambda i: (i, 0)),
        pl.BlockSpec((_G, _EP), lambda i: (i, 0)),
    ]
    w_specs = [pl.BlockSpec(w.shape, lambda i: (0,) * w.ndim) for w in weights]

    out2d = pl.pallas_call(
        _chunk_body,
        grid=grid,
        in_specs=data_specs + w_specs,
        out_specs=pl.BlockSpec((_G, _NP), lambda i: (i, 0)),
        out_shape=jax.ShapeDtypeStruct((_B, _NP), f32),
    )(mol_a_reprs, node_feats, edge_feats, src2d, dst2d, *weights)
    return out2d.reshape(-1)


# speedup vs baseline: 1.6260x; 1.0118x over previous
import jax
import jax.numpy as jnp
import numpy as np
from jax.experimental import pallas as pl

_B = 2048
_NP = 40
_EP = 120
_NF = 64
_EF = 16
_H = 128
_EH = 64
_ATT = 128
_MAX = 50
_STEPS = 3
_GS = 16
_NSUB = 8
_G = _NSUB * _GS

_INV_D = 1.0 / np.sqrt(float(_ATT))


def _bf(x):
    return x.astype(jnp.bfloat16)


def _dot(a, b):
    return jax.lax.dot_general(_bf(a), _bf(b), (((1,), (0,)), ((), ())),
                               preferred_element_type=jnp.float32)


def _dotT0(a, b):
    return jax.lax.dot_general(_bf(a), _bf(b), (((0,), (0,)), ((), ())),
                               preferred_element_type=jnp.float32)


def _dotT1(a, b):
    return jax.lax.dot_general(_bf(a), _bf(b), (((1,), (1,)), ((), ())),
                               preferred_element_type=jnp.float32)


def _chunk_body(mol_ref, nodes_ref, edges_ref, src_ref, dst_ref,
                w_ni, b_ni, w_ei, b_ei, wm_n, wm_e, b_msg, wu_h, wu_a, b_upd,
                wk1, bk1, wk2, bk2, wq1, bq1, wq2, bq2, wv1, bv1, wv2, bv2,
                uk1, ubk1, uk2, ubk2, uq1, ubq1, uq2, ubq2,
                wc1t, wc1pad, bc1, wc2, bc2,
                out_ref):
    GS = _GS
    NC = GS * _NP
    EC = GS * _EP
    f32 = jnp.float32
    relu = lambda x: jnp.maximum(x, 0.0)
    T = lambda f: tuple(f(i) for i in range(_NSUB))

    def mlp(x, w1, b1, w2, b2):
        h = T(lambda i: relu(_dot(x[i], w1[...]) + b1[...]))
        return T(lambda i: _dot(h[i], w2[...]) + b2[...])

    chunk_base = pl.program_id(0) * (_G * _NP)
    niota = jax.lax.broadcasted_iota(jnp.int32, (GS, _EP, _NP), 2)
    goff = jax.lax.broadcasted_iota(jnp.int32, (GS, 1, 1), 0) * _NP

    def bdot(a, b):
        return jax.lax.dot_general(_bf(a), _bf(b),
                                   (((2,), (1,)), ((0,), (0,))),
                                   preferred_element_type=jnp.float32)

    def bdotT(a, b):
        return jax.lax.dot_general(_bf(a), _bf(b),
                                   (((1,), (1,)), ((0,), (0,))),
                                   preferred_element_type=jnp.float32)

    def bdotR(a, b):
        return jax.lax.dot_general(_bf(a), _bf(b),
                                   (((2,), (2,)), ((0,), (0,))),
                                   preferred_element_type=jnp.float32)
    ones_bd = (jax.lax.broadcasted_iota(jnp.int32, (GS, NC), 1) // _NP ==
               jax.lax.broadcasted_iota(jnp.int32, (GS, NC), 0)).astype(f32)
    v_pad = _dot(relu(bv1[...]), wv2[...]) + bv2[...]

    src = T(lambda i: (src_ref[i * GS:(i + 1) * GS, :]
                       - (chunk_base + i * NC))[:, :, None] - goff)
    dst = T(lambda i: (dst_ref[i * GS:(i + 1) * GS, :]
                       - (chunk_base + i * NC))[:, :, None] - goff)
    oh_src = T(lambda i: (niota == src[i]).astype(jnp.bfloat16))
    oh_dst = T(lambda i: (niota == dst[i]).astype(jnp.bfloat16))

    nh = T(lambda i: relu(_dot(nodes_ref[i * NC:(i + 1) * NC, :], w_ni[...])
                          + b_ni[...]))
    ehid = T(lambda i: relu(_dot(edges_ref[i * EC:(i + 1) * EC, :], w_ei[...])
                            + b_ei[...]))
    ehc = T(lambda i: _dot(ehid[i], wm_e[...]))
    for _ in range(_STEPS):
        nhw = T(lambda i: _dot(nh[i], wm_n[...]))
        msg = T(lambda i: relu(
            bdot(oh_src[i], nhw[i].reshape(GS, _NP, _H)).reshape(EC, _H)
            + ehc[i] + b_msg[...]))
        agg = T(lambda i: bdotT(oh_dst[i],
                                msg[i].reshape(GS, _EP, _H)).reshape(NC, _H))
        nh = T(lambda i: relu(_dot(nh[i], wu_h[...])
                              + _dot(agg[i], wu_a[...]) + b_upd[...]))

    Km = mlp(nh, wk1, bk1, wk2, bk2)
    Qm = mlp(nh, wq1, bq1, wq2, bq2)
    Vm = mlp(nh, wv1, bv1, wv2, bv2)
    Vm3 = T(lambda i: Vm[i].reshape(GS, _NP, _ATT))
    kqt = T(lambda i: bdotR(Km[i].reshape(GS, _NP, _ATT),
                            Qm[i].reshape(GS, _NP, _ATT)) * _INV_D)
    m = T(lambda i: jnp.max(kqt[i], axis=2, keepdims=True))
    p = T(lambda i: jnp.exp(kqt[i] - m[i]))
    attn = T(lambda i: p[i] / jnp.sum(p[i], axis=2, keepdims=True))
    corr = T(lambda i: bdot(attn[i], Vm3[i]).reshape(NC, _ATT))

    vsum = T(lambda i: _dot(ones_bd, Vm[i]))
    corr_pad = T(lambda i: (vsum[i] + float(_MAX - _NP) * v_pad)
                 * (1.0 / _MAX))

    K2 = mlp(corr, uk1, ubk1, uk2, ubk2)
    K2p = mlp(corr_pad, uk1, ubk1, uk2, ubk2)
    Q2 = mlp(T(lambda i: mol_ref[i * GS:(i + 1) * GS, :]),
             uq1, ubq1, uq2, ubq2)
    Q2e = T(lambda i: _dotT0(ones_bd, Q2[i]))
    logit_col = T(lambda i: jnp.sum(K2[i] * Q2e[i], axis=1, keepdims=True)
                  * _INV_D)
    pad_logit = T(lambda i: jnp.sum(K2p[i] * Q2[i], axis=1, keepdims=True)
                  * _INV_D)

    h = T(lambda i: relu(_dot(ones_bd, logit_col[i] * wc1t[...])
                         + pad_logit[i] * wc1pad[...] + bc1[...]))
    o = T(lambda i: jax.nn.sigmoid(_dot(h[i], wc2[...]) + bc2[...]))
    for i in range(_NSUB):
        out_ref[i * GS:(i + 1) * GS, :] = o[i]


def kernel(mol_a_reprs, node_feats, edge_feats, edge_index, params):
    p = params
    f32 = jnp.float32
    row = lambda b: b.reshape(1, -1).astype(f32)

    src2d = edge_index[0].reshape(_B, _EP).astype(jnp.int32)
    dst2d = edge_index[1].reshape(_B, _EP).astype(jnp.int32)

    wq1, bq1, wq2, bq2 = p["Wq"]
    wk1, bk1, wk2, bk2 = p["Wk"]
    wv1, bv1, wv2, bv2 = p["Wv"]
    uk1, ubk1, uk2, ubk2 = p["Uk"]
    uq1, ubq1, uq2, ubq2 = p["Uq"]
    c1, cb1, c2, cb2 = p["Cs"]

    bf = lambda w: w.astype(jnp.bfloat16)
    weights = (
        bf(p["W_ni"]), row(p["b_ni"]), bf(p["W_ei"]), row(p["b_ei"]),
        bf(p["W_msg"][:_H]), bf(p["W_msg"][_H:]), row(p["b_msg"]),
        bf(p["W_upd"][:_H]), bf(p["W_upd"][_H:]), row(p["b_upd"]),
        bf(wk1), row(bk1), bf(wk2), row(bk2),
        bf(wq1), row(bq1), bf(wq2), row(bq2),
        bf(wv1), row(bv1), bf(wv2), row(bv2),
        bf(uk1), row(ubk1), bf(uk2), row(ubk2),
        bf(uq1), row(ubq1), bf(uq2), row(ubq2),
        jnp.tile(c1[:_NP], (_GS, 1)),
        jnp.sum(c1[_NP:], axis=0, keepdims=True),
        row(cb1), bf(c2[:, :_NP]), row(cb2[:_NP]),
    )

    grid = (_B // _G,)
    data_specs = [
        pl.BlockSpec((_G, 256), lambda i: (i, 0)),
        pl.BlockSpec((_G * _NP, _NF), lambda i: (i, 0)),
        pl.BlockSpec((_G * _EP, _EF), lambda i: (i, 0)),
        pl.BlockSpec((_G, _EP), lambda i: (i, 0)),
        pl.BlockSpec((_G, _EP), lambda i: (i, 0)),
    ]
    w_specs = [pl.BlockSpec(w.shape, lambda i: (0,) * w.ndim) for w in weights]

    out2d = pl.pallas_call(
        _chunk_body,
        grid=grid,
        in_specs=data_specs + w_specs,
        out_specs=pl.BlockSpec((_G, _NP), lambda i: (i, 0)),
        out_shape=jax.ShapeDtypeStruct((_B, _NP), f32),
    )(mol_a_reprs, node_feats, edge_feats, src2d, dst2d, *weights)
    return out2d.reshape(-1)
